# baseline (speedup 1.0000x reference)
import functools

import jax
import jax.numpy as jnp
from jax.experimental import pallas as pl
from jax.experimental.pallas import tpu as pltpu

B, N, D, E = 2, 2048, 1024, 4
H = 2 * D
EPS = 1e-5


def _ln(v, g, b):
    mu = jnp.mean(v, axis=-1, keepdims=True)
    var = jnp.mean((v - mu) ** 2, axis=-1, keepdims=True)
    return (v - mu) * jax.lax.rsqrt(var + EPS) * g + b


def _down_kernel(xn_ref, pwa_ref, pwb_ref, w_ref, b_ref, g_ref,
                 beta_ref, o_ref):
    w0 = jax.nn.sigmoid(pwa_ref[...] - pwb_ref[...])
    xg = xn_ref[...].reshape(w0.shape[0], 2, D)
    xe = xg[:, 0, :]
    xo = xg[:, 1, :]
    pooled = w0 * xe + (1.0 - w0) * xo
    xd = jnp.dot(pooled.astype(jnp.bfloat16), w_ref[...].astype(jnp.bfloat16),
                 preferred_element_type=jnp.float32) + b_ref[...]
    o_ref[...] = jnp.maximum(_ln(xd, g_ref[...], beta_ref[...]), 0.0)


def _moe_kernel(x_ref, rw_ref, rb_ref, w1_ref, b1_ref, w2_ref, b2_ref,
                g_ref, be_ref, o_ref, xb_scr, gate_scr):
    e = pl.program_id(1)

    @pl.when(e == 0)
    def _():
        xc = x_ref[...].astype(jnp.bfloat16)
        xb_scr[...] = xc
        logits = jnp.dot(xc, rw_ref[...].astype(jnp.bfloat16),
                         preferred_element_type=jnp.float32) + rb_ref[...]
        m = jnp.max(logits, axis=-1, keepdims=True)
        eg = jnp.exp(logits - m)
        gate_scr[...] = eg / jnp.sum(eg, axis=-1, keepdims=True)

    xb = xb_scr[...]
    gates = gate_scr[...]
    lane = jax.lax.broadcasted_iota(jnp.int32, gates.shape, 1)
    ge = jnp.sum(jnp.where(lane == e, gates, 0.0), axis=-1, keepdims=True)
    h2 = H // 2
    ye = b2_ref[0]
    for hh in range(2):
        w1h = w1_ref[0][:, hh * h2:(hh + 1) * h2].astype(jnp.bfloat16)
        h = jnp.dot(xb, w1h, preferred_element_type=jnp.float32)
        h = jnp.maximum(h + b1_ref[0][:, hh * h2:(hh + 1) * h2],
                        0.0).astype(jnp.bfloat16)
        w2h = w2_ref[0][hh * h2:(hh + 1) * h2, :].astype(jnp.bfloat16)
        ye = ye + jnp.dot(h, w2h, preferred_element_type=jnp.float32)
    contrib = ge * ye

    @pl.when(e == 0)
    def _():
        o_ref[...] = x_ref[...] + contrib

    @pl.when(jnp.logical_and(e > 0, e < E - 1))
    def _():
        o_ref[...] = o_ref[...] + contrib

    @pl.when(e == E - 1)
    def _():
        o_ref[...] = _ln(o_ref[...] + contrib, g_ref[...], be_ref[...])


def _up_kernel(x_ref, skn_ref, w1_ref, b1_ref, g1_ref, be1_ref,
               w2_ref, b2_ref, posp_ref, sw_ref, o_ref, w1_scr, w2_scr):
    @pl.when(pl.program_id(0) == 0)
    def _():
        w1_scr[...] = w1_ref[...].astype(jnp.bfloat16)
        w2_scr[...] = w2_ref[...].astype(jnp.bfloat16)

    t = jnp.dot(x_ref[...].astype(jnp.bfloat16), w1_scr[...],
                preferred_element_type=jnp.float32) + b1_ref[...]
    t = jnp.maximum(_ln(t, g1_ref[...], be1_ref[...]), 0.0)
    tb = t.astype(jnp.bfloat16)
    sw = sw_ref[0, 0]
    tt = tb.shape[0]
    ys = []
    for ch in range(2):
        ys.append(jnp.dot(tb, w2_scr[:, ch * D:(ch + 1) * D],
                          preferred_element_type=jnp.float32)
                  + b2_ref[:, ch * D:(ch + 1) * D]
                  + posp_ref[:, ch * D:(ch + 1) * D])
    y = jnp.stack(ys, axis=1).reshape(2 * tt, D)
    o_ref[...] = y + sw * skn_ref[...]


def _full(shape):
    nd = len(shape)
    return pl.BlockSpec(shape, lambda i, _nd=nd: (0,) * _nd)


def _rows(t, cols):
    return pl.BlockSpec((t, cols), lambda i: (i, 0))


def _downsample(xn, p, tile):
    tt = xn.shape[0] // 2
    pw = p['pool_w']
    pwa = pw[:, 0:1]
    pwb = pw[:, 1:2]
    nblk = pw.shape[0] // tile
    pwspec = pl.BlockSpec((tile, 1), lambda i, _n=nblk: (i % _n, 0))
    grid = (tt // tile,)
    return pl.pallas_call(
        _down_kernel,
        grid=grid,
        in_specs=[_rows(2 * tile, D), pwspec,
                  pwspec, _full((D, D)), _full((1, D)),
                  _full((1, D)), _full((1, D))],
        out_specs=_rows(tile, D),
        out_shape=jax.ShapeDtypeStruct((tt, D), jnp.float32),
    )(xn, pwa, pwb, p['ref_W'],
      p['ref_b'].reshape(1, D), p['ref_g'].reshape(1, D),
      p['ref_beta'].reshape(1, D))


def _moe(x, p, tile):
    tt, d = x.shape
    grid = (tt // tile, E)
    row2 = pl.BlockSpec((tile, d), lambda i, e: (i, 0))
    f2 = lambda shape: pl.BlockSpec(shape, lambda i, e: (0,) * len(shape))
    exp3 = lambda s1, s2: pl.BlockSpec((1, s1, s2), lambda i, e: (e, 0, 0))
    return pl.pallas_call(
        _moe_kernel,
        grid=grid,
        in_specs=[row2, f2((d, E)), f2((1, E)),
                  exp3(d, H), exp3(1, H), exp3(H, d),
                  exp3(1, d), f2((1, d)), f2((1, d))],
        out_specs=row2,
        out_shape=jax.ShapeDtypeStruct((tt, d), jnp.float32),
        scratch_shapes=[pltpu.VMEM((tile, d), jnp.bfloat16),
                        pltpu.VMEM((tile, E), jnp.float32)],
        compiler_params=pltpu.CompilerParams(
            vmem_limit_bytes=100 * 1024 * 1024),
    )(x, p['rW'], p['rb'].reshape(1, E),
      p['W1'], p['b1'].reshape(E, 1, H),
      p['W2'], p['b2'].reshape(E, 1, d),
      p['g'].reshape(1, d), p['be'].reshape(1, d))


def _upsample(x, skn, p, sw, tile):
    tt, d = x.shape
    grid = (tt // tile,)
    return pl.pallas_call(
        _up_kernel,
        grid=grid,
        in_specs=[_rows(tile, d), _rows(2 * tile, d),
                  _full((d, 2 * d)), _full((1, 2 * d)), _full((1, 2 * d)),
                  _full((1, 2 * d)), _full((2 * d, 2 * d)),
                  _full((1, 2 * d)), _full((1, 2 * d)), _full((1, 1))],
        out_specs=_rows(2 * tile, d),
        out_shape=jax.ShapeDtypeStruct((2 * tt, d), jnp.float32),
        scratch_shapes=[pltpu.VMEM((d, 2 * d), jnp.bfloat16),
                        pltpu.VMEM((2 * d, 2 * d), jnp.bfloat16)],
        compiler_params=pltpu.CompilerParams(
            vmem_limit_bytes=100 * 1024 * 1024),
    )(x, skn, p['W1'], p['b1'].reshape(1, 2 * d),
      p['g1'].reshape(1, 2 * d), p['be1'].reshape(1, 2 * d),
      p['W2'], p['b2'].reshape(1, 2 * d),
      p['pos'].reshape(1, 2 * d), jnp.reshape(sw, (1, 1)))


@functools.partial(jax.jit, static_argnames=())
def kernel(x, params):
    p = params
    xn = x.reshape(B * N, D)
    x1 = _downsample(xn, p['down1'], 512)
    x1 = _moe(x1, p['moe1'], 1024)
    x2 = _downsample(x1, p['down2'], 512)
    x2 = _moe(x2, p['moe2'], 1024)

    x3 = _upsample(x2, x1, p['up1'], p['sw1'], 512)
    x3 = _moe(x3, p['moe3'], 1024)

    x4 = _upsample(x3, xn, p['up2'], p['sw2'], 512)
    x4 = _moe(x4, p['moe4'], 1024)
    return x4.reshape(B, N, D)


# speedup vs baseline: 1.0042x; 1.0042x over previous
import functools

import jax
import jax.numpy as jnp
from jax.experimental import pallas as pl
from jax.experimental.pallas import tpu as pltpu

B, N, D, E = 2, 2048, 1024, 4
H = 2 * D
EPS = 1e-5


def _ln(v, g, b):
    mu = jnp.mean(v, axis=-1, keepdims=True)
    var = jnp.mean((v - mu) ** 2, axis=-1, keepdims=True)
    return (v - mu) * jax.lax.rsqrt(var + EPS) * g + b


def _down_kernel(xn_ref, pwa_ref, pwb_ref, w_ref, b_ref, g_ref,
                 beta_ref, o_ref):
    w0 = jax.nn.sigmoid(pwa_ref[...] - pwb_ref[...])
    xg = xn_ref[...].reshape(w0.shape[0], 2, D)
    xe = xg[:, 0, :]
    xo = xg[:, 1, :]
    pooled = w0 * xe + (1.0 - w0) * xo
    xd = jnp.dot(pooled.astype(jnp.bfloat16), w_ref[...].astype(jnp.bfloat16),
                 preferred_element_type=jnp.float32) + b_ref[...]
    o_ref[...] = jnp.maximum(_ln(xd, g_ref[...], beta_ref[...]), 0.0)


def _moe_kernel(x_ref, rw_ref, rb_ref, w1_ref, b1_ref, w2_ref, b2_ref,
                g_ref, be_ref, o_ref, xb_scr, gate_scr):
    e = pl.program_id(1)

    @pl.when(e == 0)
    def _():
        xc = x_ref[...].astype(jnp.bfloat16)
        xb_scr[...] = xc
        logits = jnp.dot(xc, rw_ref[...].astype(jnp.bfloat16),
                         preferred_element_type=jnp.float32) + rb_ref[...]
        m = jnp.max(logits, axis=-1, keepdims=True)
        eg = jnp.exp(logits - m)
        gate_scr[...] = eg / jnp.sum(eg, axis=-1, keepdims=True)

    xb = xb_scr[...]
    gates = gate_scr[...]
    lane = jax.lax.broadcasted_iota(jnp.int32, gates.shape, 1)
    ge = jnp.sum(jnp.where(lane == e, gates, 0.0), axis=-1, keepdims=True)
    h2 = H // 2
    ye = b2_ref[0]
    for hh in range(2):
        w1h = w1_ref[0][:, hh * h2:(hh + 1) * h2].astype(jnp.bfloat16)
        h = jnp.dot(xb, w1h, preferred_element_type=jnp.float32)
        h = jnp.maximum(h + b1_ref[0][:, hh * h2:(hh + 1) * h2],
                        0.0).astype(jnp.bfloat16)
        w2h = w2_ref[0][hh * h2:(hh + 1) * h2, :].astype(jnp.bfloat16)
        ye = ye + jnp.dot(h, w2h, preferred_element_type=jnp.float32)
    contrib = ge * ye

    @pl.when(e == 0)
    def _():
        o_ref[...] = x_ref[...] + contrib

    @pl.when(jnp.logical_and(e > 0, e < E - 1))
    def _():
        o_ref[...] = o_ref[...] + contrib

    @pl.when(e == E - 1)
    def _():
        o_ref[...] = _ln(o_ref[...] + contrib, g_ref[...], be_ref[...])


def _up_kernel(x_ref, skn_ref, w1_ref, b1_ref, g1_ref, be1_ref,
               w2_ref, b2_ref, posp_ref, sw_ref, o_ref):
    t = jnp.dot(x_ref[...].astype(jnp.bfloat16),
                w1_ref[...].astype(jnp.bfloat16),
                preferred_element_type=jnp.float32) + b1_ref[...]
    t = jnp.maximum(_ln(t, g1_ref[...], be1_ref[...]), 0.0)
    tb = t.astype(jnp.bfloat16)
    sw = sw_ref[0, 0]
    tt = tb.shape[0]
    ys = []
    for ch in range(2):
        w2h = w2_ref[:, ch * D:(ch + 1) * D].astype(jnp.bfloat16)
        ys.append(jnp.dot(tb, w2h, preferred_element_type=jnp.float32)
                  + b2_ref[:, ch * D:(ch + 1) * D]
                  + posp_ref[:, ch * D:(ch + 1) * D])
    y = jnp.stack(ys, axis=1).reshape(2 * tt, D)
    o_ref[...] = y + sw * skn_ref[...]


def _full(shape):
    nd = len(shape)
    return pl.BlockSpec(shape, lambda i, _nd=nd: (0,) * _nd)


def _rows(t, cols):
    return pl.BlockSpec((t, cols), lambda i: (i, 0))


def _downsample(xn, p, tile):
    tt = xn.shape[0] // 2
    pw = p['pool_w']
    pwa = pw[:, 0:1]
    pwb = pw[:, 1:2]
    nblk = pw.shape[0] // tile
    pwspec = pl.BlockSpec((tile, 1), lambda i, _n=nblk: (i % _n, 0))
    grid = (tt // tile,)
    return pl.pallas_call(
        _down_kernel,
        grid=grid,
        in_specs=[_rows(2 * tile, D), pwspec,
                  pwspec, _full((D, D)), _full((1, D)),
                  _full((1, D)), _full((1, D))],
        out_specs=_rows(tile, D),
        out_shape=jax.ShapeDtypeStruct((tt, D), jnp.float32),
    )(xn, pwa, pwb, p['ref_W'],
      p['ref_b'].reshape(1, D), p['ref_g'].reshape(1, D),
      p['ref_beta'].reshape(1, D))


def _moe(x, p, tile):
    tt, d = x.shape
    grid = (tt // tile, E)
    row2 = pl.BlockSpec((tile, d), lambda i, e: (i, 0))
    f2 = lambda shape: pl.BlockSpec(shape, lambda i, e: (0,) * len(shape))
    exp3 = lambda s1, s2: pl.BlockSpec((1, s1, s2), lambda i, e: (e, 0, 0))
    return pl.pallas_call(
        _moe_kernel,
        grid=grid,
        in_specs=[row2, f2((d, E)), f2((1, E)),
                  exp3(d, H), exp3(1, H), exp3(H, d),
                  exp3(1, d), f2((1, d)), f2((1, d))],
        out_specs=row2,
        out_shape=jax.ShapeDtypeStruct((tt, d), jnp.float32),
        scratch_shapes=[pltpu.VMEM((tile, d), jnp.bfloat16),
                        pltpu.VMEM((tile, E), jnp.float32)],
        compiler_params=pltpu.CompilerParams(
            vmem_limit_bytes=100 * 1024 * 1024),
    )(x, p['rW'], p['rb'].reshape(1, E),
      p['W1'], p['b1'].reshape(E, 1, H),
      p['W2'], p['b2'].reshape(E, 1, d),
      p['g'].reshape(1, d), p['be'].reshape(1, d))


def _upsample(x, skn, p, sw, tile):
    tt, d = x.shape
    grid = (tt // tile,)
    return pl.pallas_call(
        _up_kernel,
        grid=grid,
        in_specs=[_rows(tile, d), _rows(2 * tile, d),
                  _full((d, 2 * d)), _full((1, 2 * d)), _full((1, 2 * d)),
                  _full((1, 2 * d)), _full((2 * d, 2 * d)),
                  _full((1, 2 * d)), _full((1, 2 * d)), _full((1, 1))],
        out_specs=_rows(2 * tile, d),
        out_shape=jax.ShapeDtypeStruct((2 * tt, d), jnp.float32),
        compiler_params=pltpu.CompilerParams(
            vmem_limit_bytes=100 * 1024 * 1024),
    )(x, skn, p['W1'], p['b1'].reshape(1, 2 * d),
      p['g1'].reshape(1, 2 * d), p['be1'].reshape(1, 2 * d),
      p['W2'], p['b2'].reshape(1, 2 * d),
      p['pos'].reshape(1, 2 * d), jnp.reshape(sw, (1, 1)))


@functools.partial(jax.jit, static_argnames=())
def kernel(x, params):
    p = params
    xn = x.reshape(B * N, D)
    x1 = _downsample(xn, p['down1'], 512)
    x1 = _moe(x1, p['moe1'], 1024)
    x2 = _downsample(x1, p['down2'], 512)
    x2 = _moe(x2, p['moe2'], 1024)

    x3 = _upsample(x2, x1, p['up1'], p['sw1'], 512)
    x3 = _moe(x3, p['moe3'], 1024)

    x4 = _upsample(x3, xn, p['up2'], p['sw2'], 512)
    x4 = _moe(x4, p['moe4'], 1024)
    return x4.reshape(B, N, D)
